# Initial kernel scaffold; baseline (speedup 1.0000x reference)
import jax
import jax.numpy as jnp
from jax.experimental import pallas as pl


def kernel(node_species, element_indices, embed_table):
    raise NotImplementedError("write your pallas kernel here")


# speedup vs baseline: 4.0665x; 4.0665x over previous
import functools

import jax
import jax.numpy as jnp
from jax import lax
from jax.experimental import pallas as pl
from jax.experimental.pallas import tpu as pltpu
from jax.experimental.pallas import tpu_sc as plsc

N_NODES = 100000
OUT_DIM = 256
MAX_SPECIES = 119

NC, NS = 2, 16
NW = NC * NS
CHUNK = 128
FULL_CHUNKS = N_NODES // CHUNK
TAIL = N_NODES - FULL_CHUNKS * CHUNK
STEPS = -(-FULL_CHUNKS // NW)

_mesh = plsc.VectorSubcoreMesh(core_axis_name="c", subcore_axis_name="s")


CT_ROWS = 128


@functools.partial(
    pl.kernel,
    mesh=_mesh,
    out_type=jax.ShapeDtypeStruct((CT_ROWS, OUT_DIM), jnp.float32),
    scratch_types=[
        pltpu.VMEM((CT_ROWS,), jnp.int32),
        pltpu.VMEM((CT_ROWS, OUT_DIM), jnp.float32),
        pltpu.SemaphoreType.DMA,
    ],
)
def _build_ctable(elem_hbm, table_hbm, ctable_hbm, elem_v, rows_v, sem):
    wid = lax.axis_index("s") * NC + lax.axis_index("c")

    @pl.when(wid == 0)
    def _():
        elem_v[pl.ds(MAX_SPECIES - 16, 16)] = jnp.zeros((16,), jnp.int32)
        elem_v[pl.ds(CT_ROWS - 16, 16)] = jnp.zeros((16,), jnp.int32)
        pltpu.sync_copy(elem_hbm, elem_v.at[pl.ds(0, MAX_SPECIES)])
        pltpu.async_copy(table_hbm.at[elem_v], rows_v, sem).wait()
        pltpu.sync_copy(rows_v, ctable_hbm)


@functools.partial(
    pl.kernel,
    mesh=_mesh,
    out_type=jax.ShapeDtypeStruct((N_NODES, OUT_DIM), jnp.float32),
    scratch_types=[
        pltpu.VMEM((CHUNK,), jnp.int32),
        pltpu.VMEM((CHUNK, OUT_DIM), jnp.float32),
        pltpu.SemaphoreType.DMA,
    ],
)
def _sc_embed(ns_hbm, ctable_hbm, out_hbm, ns_v, rows_v, sem):
    wid = lax.axis_index("s") * NC + lax.axis_index("c")

    def do_chunk(base, rows):
        pltpu.sync_copy(ns_hbm.at[pl.ds(base, rows)], ns_v.at[pl.ds(0, rows)])
        pltpu.async_copy(ctable_hbm.at[ns_v.at[pl.ds(0, rows)]],
                         rows_v.at[pl.ds(0, rows)], sem).wait()
        pltpu.sync_copy(rows_v.at[pl.ds(0, rows)],
                        out_hbm.at[pl.ds(base, rows)])

    for g in range(STEPS):
        c = wid + g * NW

        @pl.when(c < FULL_CHUNKS)
        def _():
            do_chunk(c * CHUNK, CHUNK)

    @pl.when(wid == 0)
    def _():
        do_chunk(FULL_CHUNKS * CHUNK, TAIL)


def kernel(node_species, element_indices, embed_table):
    ctable = _build_ctable(element_indices.astype(jnp.int32), embed_table)
    return _sc_embed(node_species.astype(jnp.int32), ctable)


# speedup vs baseline: 4.0789x; 1.0031x over previous
import functools

import jax
import jax.numpy as jnp
from jax import lax
from jax.experimental import pallas as pl
from jax.experimental.pallas import tpu as pltpu
from jax.experimental.pallas import tpu_sc as plsc

N_NODES = 100000
OUT_DIM = 256
MAX_SPECIES = 119

NC, NS = 2, 16
NW = NC * NS
CHUNK = 128
FULL_CHUNKS = N_NODES // CHUNK
TAIL = N_NODES - FULL_CHUNKS * CHUNK
STEPS = -(-FULL_CHUNKS // NW)

_mesh = plsc.VectorSubcoreMesh(core_axis_name="c", subcore_axis_name="s")


CT_ROWS = 128


@functools.partial(
    pl.kernel,
    mesh=_mesh,
    out_type=jax.ShapeDtypeStruct((CT_ROWS, OUT_DIM), jnp.float32),
    scratch_types=[
        pltpu.VMEM((CT_ROWS,), jnp.int32),
        pltpu.VMEM((CT_ROWS, OUT_DIM), jnp.float32),
        pltpu.SemaphoreType.DMA,
    ],
)
def _build_ctable(elem_hbm, table_hbm, ctable_hbm, elem_v, rows_v, sem):
    wid = lax.axis_index("s") * NC + lax.axis_index("c")

    @pl.when(wid == 0)
    def _():
        elem_v[pl.ds(MAX_SPECIES - 16, 16)] = jnp.zeros((16,), jnp.int32)
        elem_v[pl.ds(CT_ROWS - 16, 16)] = jnp.zeros((16,), jnp.int32)
        pltpu.sync_copy(elem_hbm, elem_v.at[pl.ds(0, MAX_SPECIES)])
        pltpu.async_copy(table_hbm.at[elem_v], rows_v, sem).wait()
        pltpu.sync_copy(rows_v, ctable_hbm)


BASE_CH = FULL_CHUNKS // NW
EXTRA = FULL_CHUNKS - BASE_CH * NW
MAX_CH = BASE_CH + 1
NBUF = 3
IDX_CAP = MAX_CH * CHUNK


@functools.partial(
    pl.kernel,
    mesh=_mesh,
    out_type=jax.ShapeDtypeStruct((N_NODES, OUT_DIM), jnp.float32),
    scratch_types=[
        pltpu.VMEM((IDX_CAP,), jnp.int32),
        pltpu.VMEM((CHUNK, OUT_DIM), jnp.float32),
        pltpu.VMEM((CHUNK, OUT_DIM), jnp.float32),
        pltpu.VMEM((CHUNK, OUT_DIM), jnp.float32),
        pltpu.SemaphoreType.DMA,
        pltpu.SemaphoreType.DMA,
        pltpu.SemaphoreType.DMA,
        pltpu.SemaphoreType.DMA,
        pltpu.SemaphoreType.DMA,
        pltpu.SemaphoreType.DMA,
    ],
)
def _sc_embed(ns_hbm, ctable_hbm, out_hbm, idx_all,
              rows0, rows1, rows2, g0, g1, g2, w0, w1, w2):
    wid = lax.axis_index("s") * NC + lax.axis_index("c")
    rows = (rows0, rows1, rows2)
    gsem = (g0, g1, g2)
    wsem = (w0, w1, w2)

    nchunks = BASE_CH + (wid < EXTRA).astype(jnp.int32)
    start = BASE_CH * wid + jnp.minimum(wid, EXTRA)
    base_row = start * CHUNK

    pltpu.sync_copy(ns_hbm.at[pl.ds(base_row, BASE_CH * CHUNK)],
                    idx_all.at[pl.ds(0, BASE_CH * CHUNK)])

    @pl.when(wid < EXTRA)
    def _():
        pltpu.sync_copy(ns_hbm.at[pl.ds(base_row + BASE_CH * CHUNK, CHUNK)],
                        idx_all.at[pl.ds(BASE_CH * CHUNK, CHUNK)])

    def issue_gather(g, b):
        return pltpu.async_copy(
            ctable_hbm.at[idx_all.at[pl.ds(g * CHUNK, CHUNK)]], rows[b], gsem[b])

    def issue_write(g, b):
        return pltpu.async_copy(
            rows[b], out_hbm.at[pl.ds((start + g) * CHUNK, CHUNK)], wsem[b])

    n_ss = -(-MAX_CH // NBUF)
    for ss in range(n_ss):
        for j in range(NBUF):
            g = ss * NBUF + j
            if g >= MAX_CH:
                continue

            @pl.when(g < nchunks)
            def _(g=g, j=j):
                if ss > 0:
                    pltpu.make_async_copy(
                        rows[j], out_hbm.at[pl.ds(0, CHUNK)], wsem[j]).wait()
                issue_gather(g, j)

        for j in range(NBUF):
            g = ss * NBUF + j
            if g >= MAX_CH:
                continue

            @pl.when(g < nchunks)
            def _(g=g, j=j):
                pltpu.make_async_copy(ctable_hbm, rows[j], gsem[j]).wait()
                issue_write(g, j)

    for j in range(NBUF):
        pltpu.make_async_copy(rows[j], out_hbm.at[pl.ds(0, CHUNK)], wsem[j]).wait()

    @pl.when(wid == NW - 1)
    def _():
        t0 = BASE_CH * CHUNK
        pltpu.sync_copy(ns_hbm.at[pl.ds(FULL_CHUNKS * CHUNK, TAIL)],
                        idx_all.at[pl.ds(t0, TAIL)])
        pltpu.async_copy(ctable_hbm.at[idx_all.at[pl.ds(t0, TAIL)]],
                         rows0.at[pl.ds(0, TAIL)], g0).wait()
        pltpu.sync_copy(rows0.at[pl.ds(0, TAIL)],
                        out_hbm.at[pl.ds(FULL_CHUNKS * CHUNK, TAIL)])


def kernel(node_species, element_indices, embed_table):
    ctable = _build_ctable(element_indices.astype(jnp.int32), embed_table)
    return _sc_embed(node_species.astype(jnp.int32), ctable)


# speedup vs baseline: 6.8498x; 1.6793x over previous
import functools

import jax
import jax.numpy as jnp
from jax import lax
from jax.experimental import pallas as pl
from jax.experimental.pallas import tpu as pltpu
from jax.experimental.pallas import tpu_sc as plsc

N_NODES = 100000
OUT_DIM = 256
MAX_SPECIES = 119

NC, NS = 2, 16
NW = NC * NS
CHUNK = 128
FULL_CHUNKS = N_NODES // CHUNK
TAIL = N_NODES - FULL_CHUNKS * CHUNK
STEPS = -(-FULL_CHUNKS // NW)

_mesh = plsc.VectorSubcoreMesh(core_axis_name="c", subcore_axis_name="s")


CT_ROWS = 128


NREP = 32


@functools.partial(
    pl.kernel,
    mesh=_mesh,
    out_type=jax.ShapeDtypeStruct((NREP * CT_ROWS, OUT_DIM), jnp.float32),
    scratch_types=[
        pltpu.VMEM((CT_ROWS,), jnp.int32),
        pltpu.VMEM((CT_ROWS, OUT_DIM), jnp.float32),
        pltpu.SemaphoreType.DMA,
    ],
)
def _build_ctable(elem_hbm, table_hbm, ctable_hbm, elem_v, rows_v, sem):
    wid = lax.axis_index("s") * NC + lax.axis_index("c")

    @pl.when(wid < NREP)
    def _():
        elem_v[pl.ds(MAX_SPECIES - 16, 16)] = jnp.zeros((16,), jnp.int32)
        elem_v[pl.ds(CT_ROWS - 16, 16)] = jnp.zeros((16,), jnp.int32)
        pltpu.sync_copy(elem_hbm, elem_v.at[pl.ds(0, MAX_SPECIES)])
        pltpu.async_copy(table_hbm.at[elem_v], rows_v, sem).wait()
        pltpu.sync_copy(rows_v, ctable_hbm.at[pl.ds(wid * CT_ROWS, CT_ROWS)])


BASE_CH = FULL_CHUNKS // NW
EXTRA = FULL_CHUNKS - BASE_CH * NW
MAX_CH = BASE_CH + 1
NBUF = 3
IDX_CAP = MAX_CH * CHUNK


@functools.partial(
    pl.kernel,
    mesh=_mesh,
    out_type=jax.ShapeDtypeStruct((N_NODES, OUT_DIM), jnp.float32),
    scratch_types=[
        pltpu.VMEM((IDX_CAP,), jnp.int32),
        pltpu.VMEM((CHUNK, OUT_DIM), jnp.float32),
        pltpu.VMEM((CHUNK, OUT_DIM), jnp.float32),
        pltpu.VMEM((CHUNK, OUT_DIM), jnp.float32),
        pltpu.SemaphoreType.DMA,
        pltpu.SemaphoreType.DMA,
        pltpu.SemaphoreType.DMA,
        pltpu.SemaphoreType.DMA,
        pltpu.SemaphoreType.DMA,
        pltpu.SemaphoreType.DMA,
    ],
)
def _sc_embed(ns_hbm, ctable_hbm, out_hbm, idx_all,
              rows0, rows1, rows2, g0, g1, g2, w0, w1, w2):
    wid = lax.axis_index("s") * NC + lax.axis_index("c")
    rows = (rows0, rows1, rows2)
    gsem = (g0, g1, g2)
    wsem = (w0, w1, w2)

    nchunks = BASE_CH + (wid < EXTRA).astype(jnp.int32)
    start = BASE_CH * wid + jnp.minimum(wid, EXTRA)
    base_row = start * CHUNK

    pltpu.sync_copy(ns_hbm.at[pl.ds(base_row, BASE_CH * CHUNK)],
                    idx_all.at[pl.ds(0, BASE_CH * CHUNK)])

    @pl.when(wid < EXTRA)
    def _():
        pltpu.sync_copy(ns_hbm.at[pl.ds(base_row + BASE_CH * CHUNK, CHUNK)],
                        idx_all.at[pl.ds(BASE_CH * CHUNK, CHUNK)])

    off = (wid % NREP) * CT_ROWS
    for i in range(IDX_CAP // 16):
        idx_all[pl.ds(i * 16, 16)] = idx_all[pl.ds(i * 16, 16)] + off

    def issue_gather(g, b):
        return pltpu.async_copy(
            ctable_hbm.at[idx_all.at[pl.ds(g * CHUNK, CHUNK)]], rows[b], gsem[b])

    def issue_write(g, b):
        return pltpu.async_copy(
            rows[b], out_hbm.at[pl.ds((start + g) * CHUNK, CHUNK)], wsem[b])

    def drain_gather(b):
        pltpu.make_async_copy(ctable_hbm.at[pl.ds(0, CHUNK)], rows[b],
                              gsem[b]).wait()

    def drain_write(b):
        pltpu.make_async_copy(rows[b], out_hbm.at[pl.ds(0, CHUNK)],
                              wsem[b]).wait()

    for t in range(MAX_CH):

        @pl.when(t < nchunks)
        def _(t=t):
            if t >= NBUF:
                drain_write(t % NBUF)
            issue_gather(t, t % NBUF)

        if t >= 1:

            @pl.when(t - 1 < nchunks)
            def _(t=t):
                drain_gather((t - 1) % NBUF)
                issue_write(t - 1, (t - 1) % NBUF)

    @pl.when(MAX_CH - 1 < nchunks)
    def _():
        drain_gather((MAX_CH - 1) % NBUF)
        issue_write(MAX_CH - 1, (MAX_CH - 1) % NBUF)

    for j in range(NBUF):
        drain_write(j)

    @pl.when(wid == NW - 1)
    def _():
        t0 = BASE_CH * CHUNK
        pltpu.sync_copy(ns_hbm.at[pl.ds(FULL_CHUNKS * CHUNK, TAIL)],
                        idx_all.at[pl.ds(t0, TAIL)])
        pltpu.async_copy(ctable_hbm.at[idx_all.at[pl.ds(t0, TAIL)]],
                         rows0.at[pl.ds(0, TAIL)], g0).wait()
        pltpu.sync_copy(rows0.at[pl.ds(0, TAIL)],
                        out_hbm.at[pl.ds(FULL_CHUNKS * CHUNK, TAIL)])


def kernel(node_species, element_indices, embed_table):
    ctable = _build_ctable(element_indices.astype(jnp.int32), embed_table)
    return _sc_embed(node_species.astype(jnp.int32), ctable)


# speedup vs baseline: 6.8703x; 1.0030x over previous
import functools

import jax
import jax.numpy as jnp
from jax import lax
from jax.experimental import pallas as pl
from jax.experimental.pallas import tpu as pltpu
from jax.experimental.pallas import tpu_sc as plsc

N_NODES = 100000
OUT_DIM = 256
MAX_SPECIES = 119

NC, NS = 2, 16
NW = NC * NS
CHUNK = 112
FULL_CHUNKS = N_NODES // CHUNK
TAIL = N_NODES - FULL_CHUNKS * CHUNK

_mesh = plsc.VectorSubcoreMesh(core_axis_name="c", subcore_axis_name="s")


CT_ROWS = 128


NREP = 32


@functools.partial(
    pl.kernel,
    mesh=_mesh,
    out_type=jax.ShapeDtypeStruct((NREP * CT_ROWS, OUT_DIM), jnp.float32),
    scratch_types=[
        pltpu.VMEM((CT_ROWS,), jnp.int32),
        pltpu.VMEM((CT_ROWS, OUT_DIM), jnp.float32),
        pltpu.SemaphoreType.DMA,
    ],
)
def _build_ctable(elem_hbm, table_hbm, ctable_hbm, elem_v, rows_v, sem):
    wid = lax.axis_index("s") * NC + lax.axis_index("c")

    @pl.when(wid < NREP)
    def _():
        elem_v[pl.ds(MAX_SPECIES - 16, 16)] = jnp.zeros((16,), jnp.int32)
        elem_v[pl.ds(CT_ROWS - 16, 16)] = jnp.zeros((16,), jnp.int32)
        pltpu.sync_copy(elem_hbm, elem_v.at[pl.ds(0, MAX_SPECIES)])
        pltpu.async_copy(table_hbm.at[elem_v], rows_v, sem).wait()
        pltpu.sync_copy(rows_v, ctable_hbm.at[pl.ds(wid * CT_ROWS, CT_ROWS)])


BASE_CH = FULL_CHUNKS // NW
EXTRA = FULL_CHUNKS - BASE_CH * NW
MAX_CH = BASE_CH + 1
NBUF = 4
IDX_CAP = MAX_CH * CHUNK


@functools.partial(
    pl.kernel,
    mesh=_mesh,
    out_type=jax.ShapeDtypeStruct((N_NODES, OUT_DIM), jnp.float32),
    scratch_types=[
        pltpu.VMEM((IDX_CAP,), jnp.int32),
        pltpu.VMEM((CHUNK, OUT_DIM), jnp.float32),
        pltpu.VMEM((CHUNK, OUT_DIM), jnp.float32),
        pltpu.VMEM((CHUNK, OUT_DIM), jnp.float32),
        pltpu.VMEM((CHUNK, OUT_DIM), jnp.float32),
        pltpu.SemaphoreType.DMA,
        pltpu.SemaphoreType.DMA,
        pltpu.SemaphoreType.DMA,
        pltpu.SemaphoreType.DMA,
        pltpu.SemaphoreType.DMA,
        pltpu.SemaphoreType.DMA,
        pltpu.SemaphoreType.DMA,
        pltpu.SemaphoreType.DMA,
    ],
)
def _sc_embed(ns_hbm, ctable_hbm, out_hbm, idx_all,
              rows0, rows1, rows2, rows3,
              g0, g1, g2, g3, w0, w1, w2, w3):
    wid = lax.axis_index("s") * NC + lax.axis_index("c")
    rows = (rows0, rows1, rows2, rows3)
    gsem = (g0, g1, g2, g3)
    wsem = (w0, w1, w2, w3)

    nchunks = BASE_CH + (wid < EXTRA).astype(jnp.int32)
    start = BASE_CH * wid + jnp.minimum(wid, EXTRA)
    base_row = start * CHUNK

    pltpu.sync_copy(ns_hbm.at[pl.ds(base_row, BASE_CH * CHUNK)],
                    idx_all.at[pl.ds(0, BASE_CH * CHUNK)])

    @pl.when(wid < EXTRA)
    def _():
        pltpu.sync_copy(ns_hbm.at[pl.ds(base_row + BASE_CH * CHUNK, CHUNK)],
                        idx_all.at[pl.ds(BASE_CH * CHUNK, CHUNK)])

    off = (wid % NREP) * CT_ROWS
    for i in range(IDX_CAP // 16):
        idx_all[pl.ds(i * 16, 16)] = idx_all[pl.ds(i * 16, 16)] + off

    def issue_gather(g, b):
        return pltpu.async_copy(
            ctable_hbm.at[idx_all.at[pl.ds(g * CHUNK, CHUNK)]], rows[b], gsem[b])

    def issue_write(g, b):
        return pltpu.async_copy(
            rows[b], out_hbm.at[pl.ds((start + g) * CHUNK, CHUNK)], wsem[b])

    def drain_gather(b):
        pltpu.make_async_copy(ctable_hbm.at[pl.ds(0, CHUNK)], rows[b],
                              gsem[b]).wait()

    def drain_write(b):
        pltpu.make_async_copy(rows[b], out_hbm.at[pl.ds(0, CHUNK)],
                              wsem[b]).wait()

    for t in range(MAX_CH):

        @pl.when(t < nchunks)
        def _(t=t):
            if t >= NBUF:
                drain_write(t % NBUF)
            issue_gather(t, t % NBUF)

        if t >= 1:

            @pl.when(t - 1 < nchunks)
            def _(t=t):
                drain_gather((t - 1) % NBUF)
                issue_write(t - 1, (t - 1) % NBUF)

    @pl.when(MAX_CH - 1 < nchunks)
    def _():
        drain_gather((MAX_CH - 1) % NBUF)
        issue_write(MAX_CH - 1, (MAX_CH - 1) % NBUF)

    for j in range(NBUF):
        drain_write(j)

    @pl.when(wid == NW - 1)
    def _():
        t0 = BASE_CH * CHUNK
        pltpu.sync_copy(ns_hbm.at[pl.ds(FULL_CHUNKS * CHUNK, TAIL)],
                        idx_all.at[pl.ds(t0, TAIL)])
        pltpu.async_copy(ctable_hbm.at[idx_all.at[pl.ds(t0, TAIL)]],
                         rows0.at[pl.ds(0, TAIL)], g0).wait()
        pltpu.sync_copy(rows0.at[pl.ds(0, TAIL)],
                        out_hbm.at[pl.ds(FULL_CHUNKS * CHUNK, TAIL)])


def kernel(node_species, element_indices, embed_table):
    ctable = _build_ctable(element_indices.astype(jnp.int32), embed_table)
    return _sc_embed(node_species.astype(jnp.int32), ctable)
